# baseline (speedup 1.0000x reference)
import functools

import jax
import jax.numpy as jnp
from jax import lax
from jax.experimental import pallas as pl
from jax.experimental.pallas import tpu as pltpu
from jax.experimental.pallas import tpu_sc as plsc

_B = 128
_V = 100000
_RB = 8
_NBLK = _B // _RB
_PER_SC = _NBLK // 2


def _sc_relay_body(x_hbm, o_hbm, blk, gsem, ssem):
    c = lax.axis_index("c")
    s = lax.axis_index("s")

    def rows(i):
        return pl.ds((c * _PER_SC + i) * _RB, _RB)

    @pl.when(s == 0)
    def _drive():
        cps_in = [None] * _PER_SC
        cps_out = [None] * _PER_SC
        for i in range(2):
            cps_in[i] = pltpu.async_copy(x_hbm.at[rows(i), :], blk.at[i % 2],
                                         gsem.at[i % 2])
        for i in range(_PER_SC):
            sl = i % 2
            cps_in[i].wait()
            cps_out[i] = pltpu.async_copy(blk.at[sl], o_hbm.at[rows(i), :],
                                          ssem.at[sl])
            if i + 2 < _PER_SC:
                cps_out[i].wait()
                cps_in[i + 2] = pltpu.async_copy(x_hbm.at[rows(i + 2), :],
                                                 blk.at[sl], gsem.at[sl])
        for i in range(_PER_SC - 2, _PER_SC):
            cps_out[i].wait()


_sc_relay = functools.partial(
    pl.kernel,
    out_type=jax.ShapeDtypeStruct((_B, _V), jnp.float32),
    mesh=plsc.VectorSubcoreMesh(core_axis_name="c", subcore_axis_name="s"),
    compiler_params=pltpu.CompilerParams(needs_layout_passes=False),
    scratch_types=[
        pltpu.VMEM_SHARED((2, _RB, _V), jnp.float32),
        pltpu.SemaphoreType.DMA((2,)),
        pltpu.SemaphoreType.DMA((2,)),
    ],
)(_sc_relay_body)


_TC_RB = 16


def _tc_where_body(mask_ref, x_ref, o_ref):
    V = x_ref.shape[1]
    m8 = jnp.broadcast_to(mask_ref[0:1, :] != 0, (8, V))
    neg_inf = jnp.full((8, V), -jnp.inf, dtype=o_ref.dtype)
    for r in range(0, x_ref.shape[0], 8):
        o_ref[r:r + 8, :] = jnp.where(m8, neg_inf, x_ref[r:r + 8, :])


def _tc_where(logits, mask2d):
    B, V = logits.shape
    return pl.pallas_call(
        _tc_where_body,
        grid=(B // _TC_RB,),
        in_specs=[
            pl.BlockSpec((1, V), lambda i: (0, 0)),
            pl.BlockSpec((_TC_RB, V), lambda i: (i, 0)),
        ],
        out_specs=pl.BlockSpec((_TC_RB, V), lambda i: (i, 0)),
        out_shape=jax.ShapeDtypeStruct((B, V), logits.dtype),
    )(mask2d, logits)


def kernel(logits, generated_so_far, forbidden_token_mask):
    del generated_so_far
    B, V = logits.shape
    mask2d = forbidden_token_mask.astype(jnp.int8).reshape(1, V)
    n_forbidden = jnp.sum(forbidden_token_mask.astype(jnp.int32))
    return lax.cond(
        n_forbidden > 0,
        lambda: _tc_where(logits, mask2d),
        lambda: _sc_relay(logits),
    )


# speedup vs baseline: 1.6205x; 1.6205x over previous
import jax
import jax.numpy as jnp
from jax.experimental import pallas as pl
from jax.experimental.pallas import tpu as pltpu

_K = 6
_RC = 8


def _scatter_kernel(x_hbm, mask_ref, o_hbm, cap8, inbuf, outbuf,
                    in_sem, out_sem):
    B, V = x_hbm.shape
    nchunks = B // _RC
    n_forbidden = jnp.sum(mask_ref[...].astype(jnp.int32))

    @pl.when(n_forbidden > 0)
    def _apply_scatter():
        m8 = jnp.broadcast_to(mask_ref[0:1, :] != 0, (_RC, V))
        cap8[...] = jnp.where(m8, -jnp.inf, jnp.inf).astype(cap8.dtype)

        def in_copy(c, s):
            rows = pl.ds(c * _RC, _RC)
            return pltpu.make_async_copy(x_hbm.at[rows, :], inbuf.at[s],
                                         in_sem.at[s])

        def out_copy(c, s):
            rows = pl.ds(c * _RC, _RC)
            return pltpu.make_async_copy(outbuf.at[s], o_hbm.at[rows, :],
                                         out_sem.at[s])

        for c in range(min(_K, nchunks)):
            in_copy(c, c).start()

        for c in range(nchunks):
            s = c % _K
            in_copy(c, s).wait()
            if c >= _K:
                out_copy(c - _K, s).wait()
            outbuf[s] = jnp.minimum(inbuf[s], cap8[...])
            out_copy(c, s).start()
            nxt = c + _K
            if nxt < nchunks:
                in_copy(nxt, s).start()

        for c in range(max(0, nchunks - _K), nchunks):
            out_copy(c, c % _K).wait()


def kernel(logits, generated_so_far, forbidden_token_mask):
    del generated_so_far
    B, V = logits.shape
    mask2d = forbidden_token_mask.astype(jnp.int8).reshape(1, V)
    return pl.pallas_call(
        _scatter_kernel,
        in_specs=[
            pl.BlockSpec(memory_space=pltpu.MemorySpace.HBM),
            pl.BlockSpec(memory_space=pltpu.MemorySpace.VMEM),
        ],
        out_specs=pl.BlockSpec(memory_space=pltpu.MemorySpace.HBM),
        out_shape=jax.ShapeDtypeStruct((B, V), logits.dtype),
        input_output_aliases={0: 0},
        scratch_shapes=[
            pltpu.VMEM((_RC, V), logits.dtype),
            pltpu.VMEM((_K, _RC, V), logits.dtype),
            pltpu.VMEM((_K, _RC, V), logits.dtype),
            pltpu.SemaphoreType.DMA((_K,)),
            pltpu.SemaphoreType.DMA((_K,)),
        ],
    )(logits, mask2d)
